# baseline (speedup 1.0000x reference)
import jax
import jax.numpy as jnp
from jax.experimental import pallas as pl

SEQ_TILE = 512


def _add_kernel(x_ref, pos_ref, o_ref):
    o_ref[...] = x_ref[...] + pos_ref[...]


def kernel(x, pos_table):
    batch, seq, dim = x.shape
    n_seq = seq // SEQ_TILE
    return pl.pallas_call(
        _add_kernel,
        grid=(n_seq,),
        in_specs=[
            pl.BlockSpec((batch, SEQ_TILE, dim), lambda s: (0, s, 0)),
            pl.BlockSpec((SEQ_TILE, dim), lambda s: (s, 0)),
        ],
        out_specs=pl.BlockSpec((batch, SEQ_TILE, dim), lambda s: (0, s, 0)),
        out_shape=jax.ShapeDtypeStruct((batch, seq, dim), x.dtype),
    )(x, pos_table)
